# baseline (speedup 1.0000x reference)
import functools

import jax
import jax.numpy as jnp
from jax import lax
from jax.experimental import pallas as pl
from jax.experimental.pallas import tpu as pltpu
from jax.experimental.pallas import tpu_sc as plsc

N = 50000
D = 64
HALF = 32
E = 800000
NSUB = 16
NCORE = 2
EPT = E // NSUB
B = 400
NCH = EPT // B
NP = 50048
RPT = NP // NSUB
ZR = 184
GRP = B // 16

_f32 = jnp.float32
_i32 = jnp.int32


WB = 128
NWB = RPT // WB
WBR = RPT - NWB * WB


def _spmm_sc(jobs):
    nj = len(jobs)
    mesh = plsc.VectorSubcoreMesh(
        core_axis_name="c", subcore_axis_name="s",
        num_cores=NCORE, num_subcores=NSUB)

    @functools.partial(
        pl.kernel,
        out_type=[jax.ShapeDtypeStruct((2 * NP, HALF), _f32)] * nj,
        mesh=mesh,
        scratch_types=[
            pltpu.VMEM((B,), _i32), pltpu.VMEM((B,), _i32),
            pltpu.VMEM((B,), _i32), pltpu.VMEM((B,), _i32),
            pltpu.VMEM((B,), _f32), pltpu.VMEM((B,), _f32),
            pltpu.VMEM((B, HALF), _f32), pltpu.VMEM((B, HALF), _f32),
            pltpu.VMEM((2, WB), _i32),
            pltpu.VMEM((WBR,), _i32),
            pltpu.VMEM_SHARED((NP, HALF), _f32),
            pltpu.SemaphoreType.DMA, pltpu.SemaphoreType.DMA,
            pltpu.SemaphoreType.DMA, pltpu.SemaphoreType.DMA,
            pltpu.SemaphoreType.DMA, pltpu.SemaphoreType.DMA,
            pltpu.SemaphoreType.DMA, pltpu.SemaphoreType.DMA,
        ],
        compiler_params=pltpu.CompilerParams(use_tc_tiling_on_sc=False),
    )
    def batch_body(*refs):
        tables = refs[0:nj]
        idxs = refs[nj:2 * nj]
        vals = refs[2 * nj:3 * nj]
        outs = refs[3 * nj:4 * nj]
        (gi0, gi1, si0, si1, val0, val1, gath0, gath1, wbidx, wbidx2,
         acc_sh, sem_iv0, sem_iv1, sem_si0, sem_si1,
         sem_g0, sem_g1, sem_s0, sem_s1) = refs[4 * nj:]
        gi = (gi0, gi1)
        si = (si0, si1)
        val = (val0, val1)
        gath = (gath0, gath1)
        sem_iv = (sem_iv0, sem_iv1)
        sem_si = (sem_si0, sem_si1)
        sem_g = (sem_g0, sem_g1)
        sem_s = (sem_s0, sem_s1)
        c = lax.axis_index("c")
        s = lax.axis_index("s")
        zeros16 = jnp.zeros((16,), _f32)
        ebase = s * EPT

        for jb in range(nj):
            _one_spmm(jobs[jb][3], tables[jb], idxs[jb], vals[jb],
                      outs[jb], gi, si, val, gath, gath0, wbidx, wbidx2,
                      acc_sh, sem_iv, sem_si, sem_g, sem_s, c, s, zeros16,
                      ebase)

    def _one_spmm(gdim, table_hbm, idx_hbm, val_hbm, out_hbm,
                  gi, si, val, gath, gath0, wbidx, wbidx2, acc_sh,
                  sem_iv, sem_si, sem_g, sem_s, c, s, zeros16, ebase):
        sdim = 1 - gdim
        def zero_body(i, _):
            gath0[i, pl.ds(0, 16)] = zeros16
            gath0[i, pl.ds(16, 16)] = zeros16
            return 0
        lax.fori_loop(0, ZR, zero_body, 0)
        for k in range(RPT // ZR):
            pltpu.sync_copy(gath0.at[pl.ds(0, ZR)],
                            acc_sh.at[pl.ds(s * RPT + k * ZR, ZR)])
        plsc.subcore_barrier()

        gmul = 2
        goff_v = jnp.full((16,), c, _i32)

        def gv_start(ch, p):
            pltpu.async_copy(idx_hbm.at[gdim, pl.ds(ebase + ch * B, B)],
                             gi[p], sem_iv[p])
            pltpu.async_copy(val_hbm.at[pl.ds(ebase + ch * B, B)], val[p],
                             sem_iv[p])

        def gv_wait(ch, p):
            pltpu.make_async_copy(idx_hbm.at[gdim, pl.ds(ebase + ch * B, B)],
                                  gi[p], sem_iv[p]).wait()
            pltpu.make_async_copy(val_hbm.at[pl.ds(ebase + ch * B, B)], val[p],
                                  sem_iv[p]).wait()

        def si_start(ch, p):
            pltpu.async_copy(idx_hbm.at[sdim, pl.ds(ebase + ch * B, B)],
                             si[p], sem_si[p])

        def si_wait(ch, p):
            pltpu.make_async_copy(idx_hbm.at[sdim, pl.ds(ebase + ch * B, B)],
                                  si[p], sem_si[p]).wait()

        def transform(p):
            @plsc.parallel_loop(0, GRP, unroll=2)
            def _(g):
                t = gi[p][pl.ds(g * 16, 16)]
                gi[p][pl.ds(g * 16, 16)] = t * gmul + goff_v

        def g_start(p):
            pltpu.async_copy(table_hbm.at[gi[p]], gath[p], sem_g[p])

        def g_wait(p):
            pltpu.make_async_copy(table_hbm.at[gi[p]], gath[p],
                                  sem_g[p]).wait()

        def scale(p):
            gv = gath[p]
            vr = val[p]

            @plsc.parallel_loop(0, GRP, unroll=2)
            def _(g):
                e0 = g * 16
                vv = vr[pl.ds(e0, 16)]
                for j in range(16):
                    bv = jnp.broadcast_to(vv[j], (16,))
                    e = e0 + j
                    gv[e, pl.ds(0, 16)] = gv[e, pl.ds(0, 16)] * bv
                    gv[e, pl.ds(16, 16)] = gv[e, pl.ds(16, 16)] * bv

        def s_start(p):
            pltpu.async_copy(gath[p], acc_sh.at[si[p]], sem_s[p], add=True)

        def s_wait(p):
            pltpu.make_async_copy(gath[p], acc_sh.at[si[p]], sem_s[p]).wait()

        def process(ch, p, k, first=False, prefetch_pred=None):
            q = 1 - p
            g_wait(p)
            if first:
                si_start(ch + 1, q)
            else:
                def do_sw():
                    s_wait(q)
                    si_start(ch + 1, q)
                if k is None:
                    do_sw()
                else:
                    pl.when(k > 0)(do_sw)
                    pl.when(k == 0)(lambda: si_start(ch + 1, q))
            gv_wait(ch + 1, q)
            transform(q)
            g_start(q)
            scale(p)
            if prefetch_pred is None:
                gv_start(ch + 2, p)
            else:
                pl.when(prefetch_pred)(lambda: gv_start(ch + 2, p))
            si_wait(ch, p)
            s_start(p)

        gv_start(0, 0)
        si_start(0, 0)
        gv_start(1, 1)
        gv_wait(0, 0)
        transform(0)
        g_start(0)

        def pair_body(k, _):
            a = 2 * k
            process(a, 0, k)
            process(a + 1, 1, None, prefetch_pred=(k < NCH // 2 - 1))
            return 0
        lax.fori_loop(0, NCH // 2, pair_body, 0)

        g_wait(0)
        s_wait(1)
        scale(0)
        si_wait(NCH - 1, 0)
        s_start(0)
        s_wait(0)
        plsc.subcore_barrier()

        iota2 = lax.iota(_i32, 16) * 2
        rbase = s * RPT

        def wb_fill(k, p):
            r0 = rbase + k * WB
            for g in range(WB // 16):
                wbidx[p, pl.ds(g * 16, 16)] = iota2 + (2 * (r0 + g * 16) + c)

        def wb_bounce_start(k, p):
            pltpu.async_copy(acc_sh.at[pl.ds(rbase + k * WB, WB)],
                             gath0.at[pl.ds(p * WB, WB)], sem_g[p])

        def wb_bounce_wait(k, p):
            pltpu.make_async_copy(acc_sh.at[pl.ds(rbase + k * WB, WB)],
                                  gath0.at[pl.ds(p * WB, WB)], sem_g[p]).wait()

        def wb_scat_start(p):
            pltpu.async_copy(gath0.at[pl.ds(p * WB, WB)],
                             out_hbm.at[wbidx.at[p]], sem_s[p])

        def wb_scat_wait(p):
            pltpu.make_async_copy(gath0.at[pl.ds(p * WB, WB)],
                                  out_hbm.at[wbidx.at[p]], sem_s[p]).wait()

        wb_bounce_start(0, 0)
        wb_fill(0, 0)

        def wb_body(kk, _):
            ka = 2 * kk
            wb_bounce_wait(ka, 0)
            pl.when(kk > 0)(lambda: wb_scat_wait(1))
            wb_bounce_start(ka + 1, 1)
            wb_fill(ka + 1, 1)
            wb_scat_start(0)
            wb_bounce_wait(ka + 1, 1)
            wb_scat_wait(0)

            def nxt():
                wb_bounce_start(ka + 2, 0)
                wb_fill(ka + 2, 0)
            pl.when(kk < NWB // 2 - 1)(nxt)
            wb_scat_start(1)
            return 0
        lax.fori_loop(0, NWB // 2, wb_body, 0)

        r0 = rbase + NWB * WB
        pltpu.sync_copy(acc_sh.at[pl.ds(r0, WBR)], gath0.at[pl.ds(0, WBR)])
        for g0 in (0, 16, 32, 40):
            wbidx2[pl.ds(g0, 16)] = iota2 + (2 * (r0 + g0) + c)
        wb_scat_wait(1)
        pltpu.async_copy(gath0.at[pl.ds(0, WBR)], out_hbm.at[wbidx2],
                         sem_s[0]).wait()

    tabs = [j[0] for j in jobs]
    idxs = [j[1] for j in jobs]
    vls = [j[2] for j in jobs]
    return batch_body(*tabs, *idxs, *vls)


RB = 2000
NRB = N // RB


def _attention_mix(u1, u2, u3, att_mat, att_vec):
    a = jnp.dot(att_mat, att_vec.T, preferred_element_type=_f32)
    w1 = jnp.dot(u1, a, preferred_element_type=_f32)
    w2 = jnp.dot(u2, a, preferred_element_type=_f32)
    w3 = jnp.dot(u3, a, preferred_element_type=_f32)
    m = jnp.maximum(jnp.maximum(w1, w2), w3)
    e1 = jnp.exp(w1 - m)
    e2 = jnp.exp(w2 - m)
    e3 = jnp.exp(w3 - m)
    inv = 1.0 / (e1 + e2 + e3)
    return u1 * (e1 * inv) + u2 * (e2 * inv) + u3 * (e3 * inv)


def _gate(x, W, b):
    h = jnp.dot(x, W, preferred_element_type=_f32) + b
    return x * (1.0 / (1.0 + jnp.exp(-h)))


def _row_spec():
    return pl.BlockSpec((RB, D), lambda i: (i, 0))


def _full_spec(shape):
    return pl.BlockSpec(shape, lambda i: tuple(0 for _ in shape))


def _planar_spec(rb=RB):
    return pl.BlockSpec((2, rb, HALF), lambda i: (0, i, 0))


RBC = 1000
NRBC = N // RBC


def _stage_a(user_emb, W1, b1, W2, b2, W3, b3, Ws, bs, att_mat, att_vec):
    def body(x_ref, w1, bb1, w2, bb2, w3, bb3, ws, bbs, am, av,
             o1, o2, o3, os, om):
        x = x_ref[...]
        u1 = _gate(x, w1[...], bb1[...])
        u2 = _gate(x, w2[...], bb2[...])
        u3 = _gate(x, w3[...], bb3[...])
        us = _gate(x, ws[...], bbs[...])
        mixed = _attention_mix(u1, u2, u3, am[...], av[...]) + us * 0.5
        o1[...] = u1
        o2[...] = u2
        o3[...] = u3
        os[...] = us
        om[...] = mixed

    outs = [jax.ShapeDtypeStruct((N, D), _f32)] * 5
    w = _full_spec((D, D))
    b = _full_spec((1, D))
    return pl.pallas_call(
        body,
        grid=(NRB,),
        in_specs=[_row_spec(), w, b, w, b, w, b, w, b, w, _full_spec((1, D))],
        out_specs=[_row_spec()] * 5,
        out_shape=outs,
    )(user_emb, W1, b1.reshape(1, D), W2, b2.reshape(1, D),
      W3, b3.reshape(1, D), Ws, bs.reshape(1, D), att_mat, att_vec)


def _stage_b(u1, u2, u3, us, att_mat, att_vec):
    def body(r1, r2, r3, rs, am, av, om):
        om[...] = (_attention_mix(r1[...], r2[...], r3[...], am[...], av[...])
                   + rs[...] * 0.5)

    return pl.pallas_call(
        body,
        grid=(NRB,),
        in_specs=[_row_spec()] * 4 + [_full_spec((D, D)), _full_spec((1, D))],
        out_specs=_row_spec(),
        out_shape=jax.ShapeDtypeStruct((N, D), _f32),
    )(u1, u2, u3, us, att_mat, att_vec)


def _normalize(x):
    n = jnp.maximum(jnp.sqrt(jnp.sum(x * x, axis=1, keepdims=True)), 1e-12)
    return x / n


def _stage_c(u10, u20, u30, us0, u11, u21, u31, us1, i1,
             u12, u22, u32, us2, i2, item_emb, att_mat, att_vec):
    def body(r10, r20, r30, rs0, r11, r21, r31, rs1, ri1,
             r12, r22, r32, rs2, ri2, rie, am, av, ou, oi):
        u1f = r10[...] + _normalize(r11[...]) + _normalize(r12[...])
        u2f = r20[...] + _normalize(r21[...]) + _normalize(r22[...])
        u3f = r30[...] + _normalize(r31[...]) + _normalize(r32[...])
        usf = rs0[...] + _normalize(rs1[...]) + _normalize(rs2[...])
        ou[...] = _attention_mix(u1f, u2f, u3f, am[...], av[...]) + usf * 0.5
        oi[...] = rie[...] + _normalize(ri1[...]) + _normalize(ri2[...])

    row_c = pl.BlockSpec((RBC, D), lambda i: (i, 0))
    return pl.pallas_call(
        body,
        grid=(NRBC,),
        in_specs=([row_c] * 15 + [_full_spec((D, D)), _full_spec((1, D))]),
        out_specs=[row_c] * 2,
        out_shape=[jax.ShapeDtypeStruct((N, D), _f32)] * 2,
    )(u10, u20, u30, us0, u11, u21, u31, us1, i1,
      u12, u22, u32, us2, i2, item_emb, att_mat, att_vec)


def kernel(user_emb, item_emb, W_c1, b_c1, W_c2, b_c2, W_c3, b_c3,
           W_simple, b_simple, att_mat, att_vec,
           Hs_val, Hj_val, Hp_val, R_val,
           Hs_idx, Hj_idx, Hp_idx, R_idx):
    u10, u20, u30, us0, m1 = _stage_a(
        user_emb, W_c1, b_c1, W_c2, b_c2, W_c3, b_c3, W_simple, b_simple,
        att_mat, att_vec)

    def v(x):
        return x.reshape(2 * N, HALF)

    def std(o):
        return o[:2 * N].reshape(N, D)

    def spmm1(table, idx, vls, gdim):
        (out,) = _spmm_sc([(table, idx, vls, gdim)])
        return out

    u11p = spmm1(v(u10), Hs_idx, Hs_val, 1)
    u21p = spmm1(v(u20), Hj_idx, Hj_val, 1)
    u31p = spmm1(v(u30), Hp_idx, Hp_val, 1)
    i1p = spmm1(v(m1), R_idx, R_val, 0)
    us1p = spmm1(v(item_emb), R_idx, R_val, 1)

    m2 = _stage_b(std(u11p), std(u21p), std(u31p), std(us1p),
                  att_mat, att_vec)

    u12p = spmm1(u11p[:2 * N], Hs_idx, Hs_val, 1)
    u22p = spmm1(u21p[:2 * N], Hj_idx, Hj_val, 1)
    u32p = spmm1(u31p[:2 * N], Hp_idx, Hp_val, 1)
    i2p = spmm1(v(m2), R_idx, R_val, 0)
    us2p = spmm1(i1p[:2 * N], R_idx, R_val, 1)

    user_all, item_all = _stage_c(
        u10, u20, u30, us0, std(u11p), std(u21p), std(u31p), std(us1p),
        std(i1p), std(u12p), std(u22p), std(u32p), std(us2p), std(i2p),
        item_emb, att_mat, att_vec)
    return (user_all, item_all)
